# Initial kernel scaffold; baseline (speedup 1.0000x reference)
import jax
import jax.numpy as jnp
from jax.experimental import pallas as pl


def kernel(x_user, x_item, edge_index_user_buys_item, edge_index_item_rev_buys_user, edge_time_user_buys_item, edge_time_item_rev_buys_user, We, be, Wl, bl, Wr, br, gamma, beta):
    raise NotImplementedError("write your pallas kernel here")


# speedup vs baseline: 2.8886x; 2.8886x over previous
import functools

import jax
import jax.numpy as jnp
from jax import lax
from jax.experimental import pallas as pl
from jax.experimental.pallas import tpu as pltpu
from jax.experimental.pallas import tpu_sc as plsc

NC = 2
NS = 16
IW = 128
WZ = 624


def _subcore_range(s, total_rows):
  q, r = divmod(total_rows, NS)
  start = s * q + jnp.minimum(s, r)
  cnt = q + jnp.where(s < r, 1, 0)
  return start, cnt


def _split_copy(src, dst, s, n_rows):
  tail = n_rows - NS * WZ
  pltpu.sync_copy(src.at[pl.ds(s * WZ, WZ)], dst.at[pl.ds(s * WZ, WZ)])
  if tail:
    @pl.when(s == 0)
    def _():
      pltpu.sync_copy(src.at[pl.ds(NS * WZ, tail)],
                      dst.at[pl.ds(NS * WZ, tail)])


def _sc_edge_segsum(et_ui, dst_ui, et_iu, dst_iu, zeros128, n_dst):
  nrow = dst_ui.shape[0] // IW

  @functools.partial(
      pl.kernel,
      mesh=plsc.VectorSubcoreMesh(core_axis_name="c", subcore_axis_name="s"),
      out_type=(jax.ShapeDtypeStruct((n_dst, 128), jnp.float32),
                jax.ShapeDtypeStruct((n_dst, 128), jnp.float32)),
      scratch_types=[
          pltpu.VMEM((IW, 128), jnp.float32),
          pltpu.VMEM((IW,), jnp.int32),
          pltpu.VMEM_SHARED((n_dst, 128), jnp.float32),
      ],
  )
  def k(et_ui_h, d_ui_h, et_iu_h, d_iu_h, z_h, out_ui, out_iu,
        rows_v, idx_v, acc):
    c = lax.axis_index("c")
    s = lax.axis_index("s")

    _split_copy(z_h, acc, s, n_dst)
    plsc.subcore_barrier()

    start, cnt = _subcore_range(s, nrow)

    def run(et_h, d_h):
      def body(j, carry):
        row = start + j
        pltpu.sync_copy(et_h.at[pl.ds(row * IW, IW)], rows_v)
        pltpu.sync_copy(d_h.at[pl.ds(row * IW, IW)], idx_v)
        pltpu.sync_copy(rows_v, acc.at[idx_v], add=True)
        return carry
      lax.fori_loop(0, cnt, body, 0)

    @pl.when(c == 0)
    def _():
      run(et_ui_h, d_ui_h)

    @pl.when(c == 1)
    def _():
      run(et_iu_h, d_iu_h)

    plsc.subcore_barrier()

    @pl.when(c == 0)
    def _():
      _split_copy(acc, out_ui, s, n_dst)

    @pl.when(c == 1)
    def _():
      _split_copy(acc, out_iu, s, n_dst)

  return k(et_ui, dst_ui, et_iu, dst_iu, zeros128)


def _sc_feat_segsum(h_u0, h_u1, h_i0, h_i1, src_ui, dst_ui,
                    src_iu, dst_iu, zeros128, n_dst):
  nrow = src_ui.shape[0] // IW

  @functools.partial(
      pl.kernel,
      mesh=plsc.VectorSubcoreMesh(core_axis_name="c", subcore_axis_name="s"),
      out_type=(jax.ShapeDtypeStruct((n_dst, 128), jnp.float32),) * 4,
      scratch_types=[
          pltpu.VMEM((IW, 128), jnp.float32),
          pltpu.VMEM((IW,), jnp.int32),
          pltpu.VMEM((IW,), jnp.int32),
          pltpu.VMEM_SHARED((n_dst, 128), jnp.float32),
          pltpu.SemaphoreType.DMA,
      ],
  )
  def k(hu0, hu1, hi0, hi1, s_ui, d_ui, s_iu, d_iu, z_h,
        o_i0, o_i1, o_u0, o_u1, rows_v, sidx_v, didx_v, acc, sem):
    c = lax.axis_index("c")
    s = lax.axis_index("s")
    start, cnt = _subcore_range(s, nrow)

    def phase(h_h, s_h, d_h, out_h):
      _split_copy(z_h, acc, s, n_dst)
      plsc.subcore_barrier()

      def body(j, carry):
        row = start + j
        pltpu.sync_copy(s_h.at[pl.ds(row * IW, IW)], sidx_v)
        pltpu.sync_copy(d_h.at[pl.ds(row * IW, IW)], didx_v)
        pltpu.async_copy(h_h.at[sidx_v], rows_v, sem).wait()
        pltpu.sync_copy(rows_v, acc.at[didx_v], add=True)
        return carry
      lax.fori_loop(0, cnt, body, 0)
      plsc.subcore_barrier()
      _split_copy(acc, out_h, s, n_dst)
      plsc.subcore_barrier()

    @pl.when(c == 0)
    def _():
      phase(hu0, s_ui, d_ui, o_i0)
      phase(hu1, s_ui, d_ui, o_i1)

    @pl.when(c == 1)
    def _():
      phase(hi0, s_iu, d_iu, o_u0)
      phase(hi1, s_iu, d_iu, o_u1)

  return k(h_u0, h_u1, h_i0, h_i1, src_ui, dst_ui, src_iu, dst_iu,
           zeros128)


def _tc_epilogue(seg0, seg1, a, h0, h1, we_aug, wl, wr, b, g, bt, last):
  n = seg0.shape[0]
  blk = 2000
  grid = (n // blk,)

  def body(seg0_r, seg1_r, a_r, h0_r, h1_r, wea_r, wl_r, wr_r, b_r,
           g_r, bt_r, *outs):
    av = a_r[...]
    cnt = jnp.maximum(av[:, 16:17], 1.0)
    ea = jnp.dot(av, wea_r[...], preferred_element_type=jnp.float32,
                 precision=lax.Precision.HIGHEST)
    seg = jnp.concatenate([seg0_r[...], seg1_r[...]], axis=1) + ea
    agg = seg / cnt
    h = jnp.concatenate([h0_r[...], h1_r[...]], axis=1)
    y = (jnp.dot(agg, wl_r[...], preferred_element_type=jnp.float32,
                 precision=lax.Precision.HIGHEST)
         + jnp.dot(h, wr_r[...], preferred_element_type=jnp.float32,
                   precision=lax.Precision.HIGHEST)
         + b_r[...])
    mu = jnp.mean(y, axis=1, keepdims=True)
    var = jnp.mean((y - mu) ** 2, axis=1, keepdims=True)
    z = (y - mu) * lax.rsqrt(var + 1e-5) * g_r[...] + bt_r[...]
    if last:
      outs[0][...] = z
    else:
      outs[0][...] = z[:, :128]
      outs[1][...] = z[:, 128:]

  row_spec = lambda w: pl.BlockSpec((blk, w), lambda i: (i, 0))
  full_spec = lambda r, w: pl.BlockSpec((r, w), lambda i: (0, 0))
  in_specs = [row_spec(128), row_spec(128), row_spec(128), row_spec(128),
              row_spec(128), full_spec(128, 256), full_spec(256, 256),
              full_spec(256, 256), full_spec(1, 256), full_spec(1, 256),
              full_spec(1, 256)]
  if last:
    out_shape = jax.ShapeDtypeStruct((n, 256), jnp.float32)
    out_specs = row_spec(256)
  else:
    out_shape = (jax.ShapeDtypeStruct((n, 128), jnp.float32),) * 2
    out_specs = (row_spec(128), row_spec(128))

  return pl.pallas_call(
      body, grid=grid, in_specs=in_specs, out_specs=out_specs,
      out_shape=out_shape,
  )(seg0, seg1, a, h0, h1, we_aug, wl, wr, b.reshape(1, 256),
    g.reshape(1, 256), bt.reshape(1, 256))


def kernel(x_user, x_item, edge_index_user_buys_item,
           edge_index_item_rev_buys_user, edge_time_user_buys_item,
           edge_time_item_rev_buys_user, We, be, Wl, bl, Wr, br,
           gamma, beta):
  n_user, d = x_user.shape
  n_item = x_item.shape[0]
  e = edge_time_user_buys_item.shape[0]
  layers = Wl.shape[0]
  assert n_user == n_item and d == 256

  src_ui = edge_index_user_buys_item[0]
  dst_ui = edge_index_user_buys_item[1]
  src_iu = edge_index_item_rev_buys_user[0]
  dst_iu = edge_index_item_rev_buys_user[1]

  pad = jnp.concatenate(
      [jnp.ones((e, 1), jnp.float32), jnp.zeros((e, 111), jnp.float32)],
      axis=1)
  et_ui = jnp.concatenate([edge_time_user_buys_item, pad], axis=1)
  et_iu = jnp.concatenate([edge_time_item_rev_buys_user, pad], axis=1)

  zeros128 = jnp.zeros((n_item, 128), jnp.float32)

  a_ui, a_iu = _sc_edge_segsum(et_ui, dst_ui, et_iu, dst_iu, zeros128,
                               n_item)

  zpad = jnp.zeros((111, 256), jnp.float32)
  we_aug0 = jnp.concatenate([We[0], be[0][None, :], zpad], axis=0)
  we_aug1 = jnp.concatenate([We[1], be[1][None, :], zpad], axis=0)

  h_u0, h_u1 = x_user[:, :128], x_user[:, 128:]
  h_i0, h_i1 = x_item[:, :128], x_item[:, 128:]

  for l in range(layers):
    seg_i0, seg_i1, seg_u0, seg_u1 = _sc_feat_segsum(
        h_u0, h_u1, h_i0, h_i1, src_ui, dst_ui, src_iu, dst_iu,
        zeros128, n_item)
    last = l == layers - 1
    out_i = _tc_epilogue(seg_i0, seg_i1, a_ui, h_i0, h_i1, we_aug0,
                         Wl[l, 0], Wr[l, 0], bl[l, 0] + br[l, 0],
                         gamma[1], beta[1], last)
    out_u = _tc_epilogue(seg_u0, seg_u1, a_iu, h_u0, h_u1, we_aug1,
                         Wl[l, 1], Wr[l, 1], bl[l, 1] + br[l, 1],
                         gamma[0], beta[0], last)
    if last:
      return out_u, out_i
    h_i0, h_i1 = out_i
    h_u0, h_u1 = out_u
